# Initial kernel scaffold; baseline (speedup 1.0000x reference)
import jax
import jax.numpy as jnp
from jax.experimental import pallas as pl


def kernel(x, codebook, weight_sparse, keep_mask):
    raise NotImplementedError("write your pallas kernel here")


# speedup vs baseline: 6.0125x; 6.0125x over previous
import functools

import jax
import jax.numpy as jnp
from jax import lax
from jax.experimental import pallas as pl
from jax.experimental.pallas import tpu as pltpu
from jax.experimental.pallas import tpu_sc as plsc

NUM_FEAT = 1000000
N_FIELD = 26
HIDDEN = 16
BATCH = 16384
N = BATCH * N_FIELD

NC = 2
NS = 16
NW = NC * NS

CHUNK = 1024
IDX_G = CHUNK // 128
B_PER_W = N // NW
CHUNKS_PER_W = B_PER_W // CHUNK


def _sc_body(x_hbm, cb_hbm, w_hbm, m_hbm, out_hbm,
             idx_v, w_v, m_v, cb_v, sem_w, sem_m):
    wid = lax.axis_index("s") * NC + lax.axis_index("c")
    pltpu.sync_copy(cb_hbm, cb_v)

    def chunk_body(k, carry):
        chunk_id = wid * CHUNKS_PER_W + k
        gbase = chunk_id * CHUNK
        pltpu.sync_copy(x_hbm.at[chunk_id], idx_v)
        cps = []
        for g in range(IDX_G):
            dst = pl.ds(g * 128, 128)
            cps.append(pltpu.async_copy(w_hbm.at[idx_v.at[g]], w_v.at[dst], sem_w))
            cps.append(pltpu.async_copy(m_hbm.at[idx_v.at[g]], m_v.at[dst], sem_m))
        for cp in cps:
            cp.wait()
        cbrow = cb_v[chunk_id // (BATCH // CHUNK)]

        def row_body(r, c):
            w_v[r] = jnp.where(m_v[r] != 0.0, w_v[r], cbrow)
            return c

        lax.fori_loop(0, CHUNK, row_body, 0, unroll=4)
        pltpu.sync_copy(w_v, out_hbm.at[pl.ds(gbase, CHUNK)])
        return carry

    lax.fori_loop(0, CHUNKS_PER_W, chunk_body, 0)


@functools.partial(
    pl.kernel,
    out_type=jax.ShapeDtypeStruct((N, HIDDEN), jnp.float32),
    mesh=plsc.VectorSubcoreMesh(core_axis_name="c", subcore_axis_name="s",
                                num_cores=NC, num_subcores=NS),
    scratch_types=[
        pltpu.VMEM((IDX_G, 128), jnp.int32),
        pltpu.VMEM((CHUNK, HIDDEN), jnp.float32),
        pltpu.VMEM((CHUNK, HIDDEN), jnp.float32),
        pltpu.VMEM((N_FIELD, HIDDEN), jnp.float32),
        pltpu.SemaphoreType.DMA,
        pltpu.SemaphoreType.DMA,
    ],
    compiler_params=pltpu.CompilerParams(use_tc_tiling_on_sc=False),
)
def _sc_lookup(x_hbm, cb_hbm, w_hbm, m_hbm, out_hbm,
               idx_v, w_v, m_v, cb_v, sem_w, sem_m):
    _sc_body(x_hbm, cb_hbm, w_hbm, m_hbm, out_hbm,
             idx_v, w_v, m_v, cb_v, sem_w, sem_m)


def kernel(x, codebook, weight_sparse, keep_mask):
    xflat = x.reshape(N // CHUNK, IDX_G, 128)
    mask_f = keep_mask.astype(jnp.float32)
    out = _sc_lookup(xflat, codebook, weight_sparse, mask_f)
    return out.reshape(BATCH, N_FIELD, HIDDEN)


# speedup vs baseline: 8.8566x; 1.4730x over previous
import functools

import jax
import jax.numpy as jnp
from jax import lax
from jax.experimental import pallas as pl
from jax.experimental.pallas import tpu as pltpu
from jax.experimental.pallas import tpu_sc as plsc

NUM_FEAT = 1000000
N_FIELD = 26
HIDDEN = 16
BATCH = 16384
N = BATCH * N_FIELD

NC = 2
NS = 16
NW = NC * NS

CHUNK = 1024
IDX_G = CHUNK // 128
BCHUNKS = BATCH // CHUNK
NCHUNKS = N_FIELD * BCHUNKS
CHUNKS_PER_W = NCHUNKS // NW


def _sc_body(x_hbm, cb_hbm, t_hbm, out_hbm, idx_v, g_v, res_v, cb_v,
             sem_g, sem_o):
    wid = lax.axis_index("s") * NC + lax.axis_index("c")
    pltpu.sync_copy(cb_hbm, cb_v)
    lane_off = lax.iota(jnp.int32, 16) * CHUNK

    def chunk_body(k, carry):
        chunk = wid * CHUNKS_PER_W + k
        f = chunk // BCHUNKS
        bc = chunk % BCHUNKS
        b0 = bc * CHUNK
        pltpu.sync_copy(x_hbm.at[f, bc], idx_v)
        cps = []
        for g in range(IDX_G):
            cps.append(pltpu.async_copy(
                t_hbm.at[idx_v.at[g]], g_v.at[pl.ds(g * 128, 128)], sem_g))
        for cp in cps:
            cp.wait()

        def row_body(r, c_):
            i = (b0 + r) * N_FIELD + f
            crow = cb_v[i // BATCH]
            v = g_v[r]
            res = jnp.where(v != v, crow, v)
            plsc.store_scatter(res_v, [lane_off + r], res)
            return c_

        lax.fori_loop(0, CHUNK, row_body, 0, unroll=4)
        ops = []
        for h in range(HIDDEN):
            ops.append(pltpu.async_copy(
                res_v.at[pl.ds(h * CHUNK, CHUNK)],
                out_hbm.at[f, h, pl.ds(b0, CHUNK)], sem_o))
        for op in ops:
            op.wait()
        return carry

    lax.fori_loop(0, CHUNKS_PER_W, chunk_body, 0)


@functools.partial(
    pl.kernel,
    out_type=jax.ShapeDtypeStruct((N_FIELD, HIDDEN, BATCH), jnp.float32),
    mesh=plsc.VectorSubcoreMesh(core_axis_name="c", subcore_axis_name="s",
                                num_cores=NC, num_subcores=NS),
    scratch_types=[
        pltpu.VMEM((IDX_G, 128), jnp.int32),
        pltpu.VMEM((CHUNK, HIDDEN), jnp.float32),
        pltpu.VMEM((HIDDEN * CHUNK,), jnp.float32),
        pltpu.VMEM((N_FIELD, HIDDEN), jnp.float32),
        pltpu.SemaphoreType.DMA,
        pltpu.SemaphoreType.DMA,
    ],
    compiler_params=pltpu.CompilerParams(use_tc_tiling_on_sc=False,
                                         needs_layout_passes=False),
)
def _sc_lookup(x_hbm, cb_hbm, t_hbm, out_hbm, idx_v, g_v, res_v, cb_v,
               sem_g, sem_o):
    _sc_body(x_hbm, cb_hbm, t_hbm, out_hbm, idx_v, g_v, res_v, cb_v,
             sem_g, sem_o)


PK = 2048
_PACK_GRID = -(-NUM_FEAT // PK)


def _pack_body(m_ref, w_ref, out_ref):
    m = m_ref[...] != 0
    w = w_ref[...]
    comb = jnp.where(m, w, jnp.float32(jnp.nan))
    out_ref[...] = comb.T


_pack = pl.pallas_call(
    _pack_body,
    grid=(_PACK_GRID,),
    in_specs=[
        pl.BlockSpec((HIDDEN, PK), lambda i: (0, i)),
        pl.BlockSpec((HIDDEN, PK), lambda i: (0, i)),
    ],
    out_specs=pl.BlockSpec((PK, HIDDEN), lambda i: (i, 0)),
    out_shape=jax.ShapeDtypeStruct((NUM_FEAT, HIDDEN), jnp.float32),
)


def kernel(x, codebook, weight_sparse, keep_mask):
    x4 = x.T.reshape(N_FIELD, BCHUNKS, IDX_G, 128)
    mT8 = keep_mask.T.astype(jnp.int8)
    comb = _pack(mT8, weight_sparse.T)
    out_cm = _sc_lookup(x4, codebook, comb)
    return jnp.transpose(out_cm, (2, 0, 1))
